# Initial kernel scaffold; baseline (speedup 1.0000x reference)
import jax
import jax.numpy as jnp
from jax.experimental import pallas as pl


def kernel(embedding, memory_bank):
    raise NotImplementedError("write your pallas kernel here")


# speedup vs baseline: 3.3609x; 3.3609x over previous
import jax
import jax.numpy as jnp
from jax.experimental import pallas as pl
from jax.experimental.pallas import tpu as pltpu

Q = 784
D = 64
K = 100000
KB = 2048
NBLK = (K + KB - 1) // KB
NROWS = ((NBLK + 7) // 8) * 8
NEIGH = 9
BIG_I = 2**30


def _pass1(emb_ref, y_ref, score_ref, nn_ref, bq_ref, xs_ref, rmin_ref, ramin_ref):
    blk = pl.program_id(0)

    @pl.when(blk == 0)
    def _init():
        xs_ref[...] = emb_ref[...] * -2.0
        rmin_ref[...] = jnp.full((1, Q), jnp.inf, jnp.float32)
        ramin_ref[...] = jnp.zeros((1, Q), jnp.int32)

    y = y_ref[...]
    prod = jax.lax.dot_general(y, xs_ref[...], (((1,), (1,)), ((), ())),
                               preferred_element_type=jnp.float32)
    y2 = jnp.sum(y * y, axis=1, keepdims=True)
    d2 = prod + y2
    row_g = blk * KB + jax.lax.broadcasted_iota(jnp.int32, (KB, 1), 0)
    d2 = jnp.where(row_g < K, d2, jnp.inf)
    bmin = jnp.min(d2, axis=0, keepdims=True)
    rows = jax.lax.broadcasted_iota(jnp.int32, (KB, Q), 0)
    bamin = jnp.min(jnp.where(d2 == bmin, rows, BIG_I), axis=0, keepdims=True) + blk * KB
    upd = bmin < rmin_ref[...]
    rmin_ref[...] = jnp.where(upd, bmin, rmin_ref[...])
    ramin_ref[...] = jnp.where(upd, bamin, ramin_ref[...])

    @pl.when(blk == NBLK - 1)
    def _fin():
        e = emb_ref[...]
        ones = jnp.ones((1, D), jnp.float32)
        x2 = jax.lax.dot_general(ones, e * e, (((1,), (1,)), ((), ())),
                                 preferred_element_type=jnp.float32)
        scores = jnp.sqrt(jnp.clip(rmin_ref[...] + x2, 1e-12, None))
        smax = jnp.max(scores)
        lanes = jax.lax.broadcasted_iota(jnp.int32, (1, Q), 1)
        bq = jnp.min(jnp.where(scores == smax, lanes, BIG_I))
        nn = jnp.sum(jnp.where(lanes == bq, ramin_ref[...], 0))
        score_ref[...] = smax.reshape(1, 1)
        nn_ref[...] = nn.reshape(1, 1)
        bq_ref[...] = bq.reshape(1, 1)


def _pass2(nn_idx_ref, bq_ref, nnblk_ref, eblk_ref, y_ref, score_ref,
           out_ref, dnn_ref, dmx_ref):
    blk = pl.program_id(0)
    nn = nn_idx_ref[0]
    bq = bq_ref[0]

    @pl.when(blk == 0)
    def _init():
        dnn_ref[...] = jnp.full((NROWS, KB), jnp.inf, jnp.float32)
        dmx_ref[...] = jnp.zeros((NROWS, KB), jnp.float32)

    sub8 = jax.lax.broadcasted_iota(jnp.int32, (8, 1), 0)
    nn_row = jnp.sum(jnp.where(sub8 == nn % 8, nnblk_ref[...], 0.0),
                     axis=0, keepdims=True)
    mf_row = jnp.sum(jnp.where(sub8 == bq % 8, eblk_ref[...], 0.0),
                     axis=0, keepdims=True)
    x2m = jnp.sum(mf_row * mf_row)

    y = y_ref[...]
    rhs = jnp.concatenate([nn_row, mf_row], axis=0)
    prod = jax.lax.dot_general(rhs, y, (((1,), (1,)), ((), ())),
                               preferred_element_type=jnp.float32)
    ones = jnp.ones((1, D), jnp.float32)
    y2 = jax.lax.dot_general(ones, y * y, (((1,), (1,)), ((), ())),
                             preferred_element_type=jnp.float32)
    col_g = blk * KB + jax.lax.broadcasted_iota(jnp.int32, (1, KB), 1)
    dnn = jnp.where(col_g < K, y2 - 2.0 * prod[0:1, :], jnp.inf)
    dmx = x2m + y2 - 2.0 * prod[1:2, :]
    dnn_ref[pl.ds(blk, 1), :] = dnn
    dmx_ref[pl.ds(blk, 1), :] = dmx

    @pl.when(blk == NBLK - 1)
    def _fin():
        dnnS = dnn_ref[...]
        dmxS = dmx_ref[...]
        srow = jax.lax.broadcasted_iota(jnp.int32, (NROWS, KB), 0)
        scol = jax.lax.broadcasted_iota(jnp.int32, (NROWS, KB), 1)
        gidx = srow * KB + scol
        lanes = jax.lax.broadcasted_iota(jnp.int32, (1, 128), 1)
        vals = jnp.zeros((1, 128), jnp.float32)
        for j in range(NEIGH):
            m = jnp.min(dnnS)
            sel = jnp.min(jnp.where(dnnS == m, gidx, BIG_I))
            hit = gidx == sel
            dj = jnp.sqrt(jnp.clip(jnp.sum(jnp.where(hit, dmxS, 0.0)), 1e-12, None))
            vals = vals + jnp.where(lanes == j, dj, 0.0)
            dnnS = jnp.where(hit, jnp.inf, dnnS)
        valid9 = lanes < NEIGH
        mx = jnp.max(jnp.where(valid9, vals, -jnp.inf))
        e = jnp.where(valid9, jnp.exp(vals - mx), 0.0)
        den = jnp.sum(e)
        e0 = jnp.sum(jnp.where(lanes == 0, e, 0.0))
        w = 1.0 - e0 / den
        out_ref[...] = (w * score_ref[0, 0]).reshape(1, 1)


def kernel(embedding, memory_bank):
    score, nn_idx, bq = pl.pallas_call(
        _pass1,
        grid=(NBLK,),
        in_specs=[
            pl.BlockSpec((Q, D), lambda i: (0, 0)),
            pl.BlockSpec((KB, D), lambda i: (i, 0)),
        ],
        out_specs=[
            pl.BlockSpec((1, 1), lambda i: (0, 0)),
            pl.BlockSpec((1, 1), lambda i: (0, 0)),
            pl.BlockSpec((1, 1), lambda i: (0, 0)),
        ],
        out_shape=[
            jax.ShapeDtypeStruct((1, 1), jnp.float32),
            jax.ShapeDtypeStruct((1, 1), jnp.int32),
            jax.ShapeDtypeStruct((1, 1), jnp.int32),
        ],
        scratch_shapes=[
            pltpu.VMEM((Q, D), jnp.float32),
            pltpu.VMEM((1, Q), jnp.float32),
            pltpu.VMEM((1, Q), jnp.int32),
        ],
    )(embedding, memory_bank)

    out = pl.pallas_call(
        _pass2,
        grid_spec=pltpu.PrefetchScalarGridSpec(
            num_scalar_prefetch=2,
            grid=(NBLK,),
            in_specs=[
                pl.BlockSpec((8, D), lambda i, nn, bq: (nn[0] // 8, 0)),
                pl.BlockSpec((8, D), lambda i, nn, bq: (bq[0] // 8, 0)),
                pl.BlockSpec((KB, D), lambda i, nn, bq: (i, 0)),
                pl.BlockSpec((1, 1), lambda i, nn, bq: (0, 0)),
            ],
            out_specs=pl.BlockSpec((1, 1), lambda i, nn, bq: (0, 0)),
            scratch_shapes=[
                pltpu.VMEM((NROWS, KB), jnp.float32),
                pltpu.VMEM((NROWS, KB), jnp.float32),
            ],
        ),
        out_shape=jax.ShapeDtypeStruct((1, 1), jnp.float32),
    )(nn_idx.reshape(1), bq.reshape(1), memory_bank, embedding, memory_bank, score)

    return out.reshape(1)


# speedup vs baseline: 4.8494x; 1.4429x over previous
import jax
import jax.numpy as jnp
from jax.experimental import pallas as pl
from jax.experimental.pallas import tpu as pltpu

Q = 784
D = 64
K = 100000
KB = 2000
NBLK = K // KB
NROWS = ((NBLK + 7) // 8) * 8
NEIGH = 9
BIG_I = 2**30


def _pass1(emb_ref, y_ref, score_ref, bq_ref, bstar_ref, xs_ref, bmin_ref):
    blk = pl.program_id(0)

    @pl.when(blk == 0)
    def _init():
        xs_ref[...] = emb_ref[...] * -2.0
        bmin_ref[...] = jnp.full((NROWS, Q), jnp.inf, jnp.float32)

    y = y_ref[...]
    prod = jax.lax.dot_general(y, xs_ref[...], (((1,), (1,)), ((), ())),
                               preferred_element_type=jnp.float32)
    y2 = jnp.sum(y * y, axis=1, keepdims=True)
    d2 = prod + y2
    bmin_ref[pl.ds(blk, 1), :] = jnp.min(d2, axis=0, keepdims=True)

    @pl.when(blk == NBLK - 1)
    def _fin():
        e = emb_ref[...]
        ones = jnp.ones((1, D), jnp.float32)
        x2 = jax.lax.dot_general(ones, e * e, (((1,), (1,)), ((), ())),
                                 preferred_element_type=jnp.float32)
        allmin = bmin_ref[...]
        rmin = jnp.min(allmin, axis=0, keepdims=True)
        scores = jnp.sqrt(jnp.clip(rmin + x2, 1e-12, None))
        smax = jnp.max(scores)
        lanes = jax.lax.broadcasted_iota(jnp.int32, (1, Q), 1)
        bq = jnp.min(jnp.where(scores == smax, lanes, BIG_I))
        vstar = jnp.sum(jnp.where(lanes == bq, rmin, 0.0))
        rows = jax.lax.broadcasted_iota(jnp.int32, (NROWS, Q), 0)
        hitcol = jnp.logical_and(lanes == bq, allmin == vstar)
        bstar = jnp.min(jnp.where(hitcol, rows, BIG_I))
        score_ref[...] = smax.reshape(1, 1)
        bq_ref[...] = bq.reshape(1, 1)
        bstar_ref[...] = bstar.reshape(1, 1)


def _pass1b(bstar_ref, bq_ref, eblk_ref, y_ref, nn_ref):
    bq = bq_ref[0]
    sub8 = jax.lax.broadcasted_iota(jnp.int32, (8, 1), 0)
    xrow = jnp.sum(jnp.where(sub8 == bq % 8, eblk_ref[...], 0.0),
                   axis=0, keepdims=True)
    y = y_ref[...]
    prod = jax.lax.dot_general(y, xrow * -2.0, (((1,), (1,)), ((), ())),
                               preferred_element_type=jnp.float32)
    d2c = prod + jnp.sum(y * y, axis=1, keepdims=True)
    m = jnp.min(d2c)
    rows = jax.lax.broadcasted_iota(jnp.int32, (KB, 1), 0)
    nn_local = jnp.min(jnp.where(d2c == m, rows, BIG_I))
    nn_ref[...] = (bstar_ref[0] * KB + nn_local).reshape(1, 1)


def _pass2(nn_idx_ref, bq_ref, nnblk_ref, eblk_ref, y_ref, score_ref,
           out_ref, dnn_ref, dmx_ref):
    blk = pl.program_id(0)
    nn = nn_idx_ref[0]
    bq = bq_ref[0]

    @pl.when(blk == 0)
    def _init():
        dnn_ref[...] = jnp.full((NROWS, KB), jnp.inf, jnp.float32)
        dmx_ref[...] = jnp.zeros((NROWS, KB), jnp.float32)

    sub8 = jax.lax.broadcasted_iota(jnp.int32, (8, 1), 0)
    nn_row = jnp.sum(jnp.where(sub8 == nn % 8, nnblk_ref[...], 0.0),
                     axis=0, keepdims=True)
    mf_row = jnp.sum(jnp.where(sub8 == bq % 8, eblk_ref[...], 0.0),
                     axis=0, keepdims=True)
    x2m = jnp.sum(mf_row * mf_row)

    y = y_ref[...]
    rhs = jnp.concatenate([nn_row, mf_row], axis=0)
    prod = jax.lax.dot_general(rhs, y, (((1,), (1,)), ((), ())),
                               preferred_element_type=jnp.float32)
    ones = jnp.ones((1, D), jnp.float32)
    y2 = jax.lax.dot_general(ones, y * y, (((1,), (1,)), ((), ())),
                             preferred_element_type=jnp.float32)
    dnn_ref[pl.ds(blk, 1), :] = y2 - 2.0 * prod[0:1, :]
    dmx_ref[pl.ds(blk, 1), :] = x2m + y2 - 2.0 * prod[1:2, :]

    @pl.when(blk == NBLK - 1)
    def _fin():
        dnnS = dnn_ref[...]
        dmxS = dmx_ref[...]
        srow = jax.lax.broadcasted_iota(jnp.int32, (NROWS, KB), 0)
        scol = jax.lax.broadcasted_iota(jnp.int32, (NROWS, KB), 1)
        gidx = srow * KB + scol
        lanes = jax.lax.broadcasted_iota(jnp.int32, (1, 128), 1)
        vals = jnp.zeros((1, 128), jnp.float32)
        for j in range(NEIGH):
            m = jnp.min(dnnS)
            sel = jnp.min(jnp.where(dnnS == m, gidx, BIG_I))
            hit = gidx == sel
            dj = jnp.sqrt(jnp.clip(jnp.sum(jnp.where(hit, dmxS, 0.0)), 1e-12, None))
            vals = vals + jnp.where(lanes == j, dj, 0.0)
            dnnS = jnp.where(hit, jnp.inf, dnnS)
        valid9 = lanes < NEIGH
        mx = jnp.max(jnp.where(valid9, vals, -jnp.inf))
        e = jnp.where(valid9, jnp.exp(vals - mx), 0.0)
        den = jnp.sum(e)
        e0 = jnp.sum(jnp.where(lanes == 0, e, 0.0))
        w = 1.0 - e0 / den
        out_ref[...] = (w * score_ref[0, 0]).reshape(1, 1)


def kernel(embedding, memory_bank):
    score, bq, bstar = pl.pallas_call(
        _pass1,
        grid=(NBLK,),
        in_specs=[
            pl.BlockSpec((Q, D), lambda i: (0, 0)),
            pl.BlockSpec((KB, D), lambda i: (i, 0)),
        ],
        out_specs=[
            pl.BlockSpec((1, 1), lambda i: (0, 0)),
            pl.BlockSpec((1, 1), lambda i: (0, 0)),
            pl.BlockSpec((1, 1), lambda i: (0, 0)),
        ],
        out_shape=[
            jax.ShapeDtypeStruct((1, 1), jnp.float32),
            jax.ShapeDtypeStruct((1, 1), jnp.int32),
            jax.ShapeDtypeStruct((1, 1), jnp.int32),
        ],
        scratch_shapes=[
            pltpu.VMEM((Q, D), jnp.float32),
            pltpu.VMEM((NROWS, Q), jnp.float32),
        ],
    )(embedding, memory_bank)

    nn_idx = pl.pallas_call(
        _pass1b,
        grid_spec=pltpu.PrefetchScalarGridSpec(
            num_scalar_prefetch=2,
            grid=(1,),
            in_specs=[
                pl.BlockSpec((8, D), lambda i, bs, bq: (bq[0] // 8, 0)),
                pl.BlockSpec((KB, D), lambda i, bs, bq: (bs[0], 0)),
            ],
            out_specs=pl.BlockSpec((1, 1), lambda i, bs, bq: (0, 0)),
        ),
        out_shape=jax.ShapeDtypeStruct((1, 1), jnp.int32),
    )(bstar.reshape(1), bq.reshape(1), embedding, memory_bank)

    out = pl.pallas_call(
        _pass2,
        grid_spec=pltpu.PrefetchScalarGridSpec(
            num_scalar_prefetch=2,
            grid=(NBLK,),
            in_specs=[
                pl.BlockSpec((8, D), lambda i, nn, bq: (nn[0] // 8, 0)),
                pl.BlockSpec((8, D), lambda i, nn, bq: (bq[0] // 8, 0)),
                pl.BlockSpec((KB, D), lambda i, nn, bq: (i, 0)),
                pl.BlockSpec((1, 1), lambda i, nn, bq: (0, 0)),
            ],
            out_specs=pl.BlockSpec((1, 1), lambda i, nn, bq: (0, 0)),
            scratch_shapes=[
                pltpu.VMEM((NROWS, KB), jnp.float32),
                pltpu.VMEM((NROWS, KB), jnp.float32),
            ],
        ),
        out_shape=jax.ShapeDtypeStruct((1, 1), jnp.float32),
    )(nn_idx.reshape(1), bq.reshape(1), memory_bank, embedding, memory_bank, score)

    return out.reshape(1)
